# baseline (speedup 1.0000x reference)
import functools
import jax
import jax.numpy as jnp
from jax import lax
from jax.experimental import pallas as pl
from jax.experimental.pallas import tpu as pltpu
from jax.experimental.pallas import tpu_sc as plsc

_N, _S, _D = 4, 4096, 1024
_NC, _NS = 2, 16
_NW = _NC * _NS
_SPW = _S // _NW
_C = 32
_L = 16
_VPR = _D // _L

_mesh = plsc.VectorSubcoreMesh(
    core_axis_name="c", subcore_axis_name="s", num_cores=_NC, num_subcores=_NS
)


@functools.partial(
    pl.kernel,
    out_type=jax.ShapeDtypeStruct((_N, _S, _D), jnp.float32),
    mesh=_mesh,
    scratch_types=[
        pltpu.VMEM((_C, _D), jnp.float32),
        pltpu.VMEM((_C, _D), jnp.float32),
    ],
)
def _sc_add(x_hbm, enc_hbm, out_hbm, e_v, b_v):
    wid = lax.axis_index("s") * _NC + lax.axis_index("c")
    s_base = wid * _SPW

    def chunk_body(ci, carry):
        s0 = s_base + ci * _C
        pltpu.sync_copy(enc_hbm.at[pl.ds(s0, _C)], e_v)
        for n in range(_N):
            pltpu.sync_copy(x_hbm.at[n, pl.ds(s0, _C)], b_v)

            def row_body(r, c2):
                for i in range(_VPR):
                    vec = e_v[r, pl.ds(i * _L, _L)]
                    plsc.addupdate(b_v.at[r, pl.ds(i * _L, _L)], vec)
                return c2

            lax.fori_loop(0, _C, row_body, 0)
            pltpu.sync_copy(b_v, out_hbm.at[n, pl.ds(s0, _C)])
        return carry

    lax.fori_loop(0, _SPW // _C, chunk_body, 0)


def kernel(x, encoding):
    return _sc_add(x, encoding)


# speedup vs baseline: 1.2134x; 1.2134x over previous
import functools
import jax
import jax.numpy as jnp
from jax import lax
from jax.experimental import pallas as pl
from jax.experimental.pallas import tpu as pltpu
from jax.experimental.pallas import tpu_sc as plsc

_N, _S, _D = 4, 4096, 1024
_NC, _NS = 2, 16
_NW = _NC * _NS
_SPW = _S // _NW
_C = 16
_NCHUNK = _SPW // _C
_L = 16
_VPR = _D // _L

_mesh = plsc.VectorSubcoreMesh(
    core_axis_name="c", subcore_axis_name="s", num_cores=_NC, num_subcores=_NS
)


def _accum(b_v, e_v):
    def row_body(r, carry):
        for i in range(_VPR):
            vec = e_v[r, pl.ds(i * _L, _L)]
            plsc.addupdate(b_v.at[r, pl.ds(i * _L, _L)], vec)
        return carry

    lax.fori_loop(0, _C, row_body, 0)


@functools.partial(
    pl.kernel,
    out_type=jax.ShapeDtypeStruct((_N, _S, _D), jnp.float32),
    mesh=_mesh,
    scratch_types=[
        pltpu.VMEM((_C, _D), jnp.float32),
        pltpu.VMEM((_C, _D), jnp.float32),
        pltpu.VMEM((_C, _D), jnp.float32),
        pltpu.VMEM((_C, _D), jnp.float32),
        pltpu.SemaphoreType.DMA,
        pltpu.SemaphoreType.DMA,
        pltpu.SemaphoreType.DMA,
        pltpu.SemaphoreType.DMA,
        pltpu.SemaphoreType.DMA,
        pltpu.SemaphoreType.DMA,
    ],
)
def _sc_add(x_hbm, enc_hbm, out_hbm, e0, e1, b0, b1, er0, er1, xr0, xr1,
            wr0, wr1):
    wid = lax.axis_index("s") * _NC + lax.axis_index("c")
    s_base = wid * _SPW
    e_bufs, e_sems = (e0, e1), (er0, er1)
    x_bufs, x_sems = (b0, b1), (xr0, xr1)
    w_sems = (wr0, wr1)

    def enc_src(ci):
        return enc_hbm.at[pl.ds(s_base + ci * _C, _C)]

    def x_src(ci, n):
        return x_hbm.at[n, pl.ds(s_base + ci * _C, _C)]

    def out_dst(ci, n):
        return out_hbm.at[n, pl.ds(s_base + ci * _C, _C)]

    pltpu.async_copy(enc_src(0), e0, er0)
    pltpu.async_copy(x_src(0, 0), b0, xr0)

    def pair_body(p, carry):
        for j in range(8):
            eb = j // 4
            xb = j % 2
            ci = 2 * p + j // 4
            n = j % 4

            if j == 0:
                pltpu.make_async_copy(enc_src(2 * p), e0, er0).wait()
                pltpu.async_copy(enc_src(2 * p + 1), e1, er1)
            if j == 4:
                pltpu.make_async_copy(enc_src(2 * p + 1), e1, er1).wait()

                @pl.when(p < _NCHUNK // 2 - 1)
                def _():
                    pltpu.async_copy(enc_src(2 * p + 2), e0, er0)

            if j < 7:
                nxt = j + 1
                ci_n, n_n = 2 * p + nxt // 4, nxt % 4
                if j == 0:
                    @pl.when(p > 0)
                    def _():
                        pltpu.make_async_copy(
                            x_bufs[1], out_dst(ci, n), w_sems[1]
                        ).wait()
                else:
                    pltpu.make_async_copy(
                        x_bufs[nxt % 2], out_dst(ci, n), w_sems[nxt % 2]
                    ).wait()
                pltpu.async_copy(
                    x_src(ci_n, n_n), x_bufs[nxt % 2], x_sems[nxt % 2]
                )
            else:
                @pl.when(p < _NCHUNK // 2 - 1)
                def _():
                    pltpu.make_async_copy(
                        x_bufs[0], out_dst(ci, n), w_sems[0]
                    ).wait()
                    pltpu.async_copy(
                        x_src(2 * p + 2, 0), x_bufs[0], x_sems[0]
                    )

            pltpu.make_async_copy(x_src(ci, n), x_bufs[xb], x_sems[xb]).wait()
            _accum(x_bufs[xb], e_bufs[eb])
            pltpu.async_copy(x_bufs[xb], out_dst(ci, n), w_sems[xb])
        return carry

    lax.fori_loop(0, _NCHUNK // 2, pair_body, 0)

    last = _NCHUNK - 1
    pltpu.make_async_copy(b0, out_dst(last, 2), wr0).wait()
    pltpu.make_async_copy(b1, out_dst(last, 3), wr1).wait()


def kernel(x, encoding):
    return _sc_add(x, encoding)
